# baseline (speedup 1.0000x reference)
import functools

import jax
import jax.numpy as jnp
from jax import lax
from jax.experimental import pallas as pl
from jax.experimental.pallas import tpu as pltpu
from jax.experimental.pallas import tpu_sc as plsc

_LANES = 16


def _build_sc_kernel(B, L, C, P, S, NC, NS):
    NW = NC * NS
    PW = P // NW
    CHUNK = 4
    NCH = PW // CHUNK
    WROWS = CHUNK * S
    CVECS = C // _LANES
    QCH = _LANES // CHUNK

    @functools.partial(
        pl.kernel,
        mesh=plsc.VectorSubcoreMesh(core_axis_name="c", subcore_axis_name="s"),
        compiler_params=pltpu.CompilerParams(needs_layout_passes=False),
        out_type=jax.ShapeDtypeStruct((B, P), jnp.float32),
        scratch_types=[
            pltpu.VMEM((NCH, WROWS), jnp.int32),
            pltpu.VMEM((2, CHUNK * S, C), jnp.float32),
            pltpu.VMEM((2, B * WROWS, C), jnp.float32),
            pltpu.VMEM((PW,), jnp.float32),
            pltpu.VMEM((B, PW), jnp.float32),
            pltpu.SemaphoreType.DMA,
            pltpu.SemaphoreType.DMA,
        ],
    )
    def run(gidx_h, w_h, bias_h, y_h, out_h,
            idx_v, w_buf, rows_buf, bias_v, out_v, sem_a, sem_b):
        cid = lax.axis_index("c")
        sid = lax.axis_index("s")
        wid = sid * NC + cid

        pltpu.sync_copy(gidx_h.at[pl.ds(wid * NCH, NCH)], idx_v)
        pltpu.sync_copy(bias_h.at[pl.ds(wid * PW, PW)], bias_v)

        sems = (sem_a, sem_b)

        def issue(ch, slot):
            for p in range(CHUNK):
                pltpu.async_copy(
                    w_h.at[(wid * NCH + ch) * CHUNK + p],
                    w_buf.at[slot].at[pl.ds(p * S, S)],
                    sems[slot])
            for b in range(B):
                pltpu.async_copy(
                    y_h.at[b].at[idx_v.at[ch]],
                    rows_buf.at[slot].at[pl.ds(b * WROWS, WROWS)],
                    sems[slot])

        def wait_chunk(ch, slot):
            for p in range(CHUNK):
                pltpu.make_async_copy(
                    w_h.at[(wid * NCH + ch) * CHUNK + p],
                    w_buf.at[slot].at[pl.ds(p * S, S)],
                    sems[slot]).wait()
            for b in range(B):
                pltpu.make_async_copy(
                    y_h.at[b].at[idx_v.at[ch]],
                    rows_buf.at[slot].at[pl.ds(b * WROWS, WROWS)],
                    sems[slot]).wait()

        iota16 = lax.iota(jnp.int32, _LANES)
        zeros = jnp.zeros((_LANES,), jnp.float32)

        issue(0, 0)

        def quad_body(quad, _):
            packs = (zeros,) * B
            for ch4 in range(QCH):
                ch = quad * QCH + ch4
                slot = ch4 % 2

                @pl.when(ch + 1 < NCH)
                def _():
                    issue(ch + 1, 1 - slot)

                wait_chunk(ch, slot)

                def patch_loop(p, packs, slot=slot, ch4=ch4):
                    def ss_loop(ss, accs):
                        row = p * S + ss

                        def cc_loop(cc, accs):
                            off = pl.ds(cc * _LANES, _LANES)
                            w = w_buf[slot, row, off]
                            return tuple(
                                accs[b] + w * rows_buf[slot, b * WROWS + row, off]
                                for b in range(B)
                            )

                        return lax.fori_loop(0, CVECS, cc_loop, accs,
                                             unroll=8)

                    accs = lax.fori_loop(0, S, ss_loop, (zeros,) * B)
                    lane = ch4 * CHUNK + p
                    return tuple(
                        jnp.where(iota16 == lane,
                                  lax.broadcast(jnp.sum(accs[b]), (_LANES,)),
                                  packs[b])
                        for b in range(B)
                    )

                packs = lax.fori_loop(0, CHUNK, patch_loop, packs)

            bseg = bias_v[pl.ds(quad * _LANES, _LANES)]
            for b in range(B):
                o = packs[b] + bseg
                out_v[b, pl.ds(quad * _LANES, _LANES)] = (
                    jnp.where(o >= 0.0, o, 0.1 * o))
            return 0

        lax.fori_loop(0, NCH // QCH, quad_body, 0)

        for b in range(B):
            pltpu.sync_copy(out_v.at[b], out_h.at[b].at[pl.ds(wid * PW, PW)])

    return run


def kernel(y, patches, W_MULT, W_BIAS):
    B, L, C = y.shape
    P, S, _ = patches.shape

    info = plsc.get_sparse_core_info()
    NC, NS = info.num_cores, info.num_subcores

    w2 = W_MULT.reshape(P, S, C)
    gidx = patches[:, :, 0].astype(jnp.int32).reshape(P // 4, 4 * S)
    bias = W_BIAS.reshape(P)

    run = _build_sc_kernel(B, L, C, P, S, NC, NS)
    return run(gidx, w2, bias, y)


# speedup vs baseline: 1.0117x; 1.0117x over previous
import functools

import jax
import jax.numpy as jnp
from jax import lax
from jax.experimental import pallas as pl
from jax.experimental.pallas import tpu as pltpu
from jax.experimental.pallas import tpu_sc as plsc

_LANES = 16


def _build_sc_kernel(B, L, C, P, S, NC, NS):
    NW = NC * NS
    PW = P // NW
    CHUNK = 4
    NCH = PW // CHUNK
    WROWS = CHUNK * S
    CVECS = C // _LANES
    QCH = _LANES // CHUNK

    @functools.partial(
        pl.kernel,
        mesh=plsc.VectorSubcoreMesh(core_axis_name="c", subcore_axis_name="s"),
        compiler_params=pltpu.CompilerParams(needs_layout_passes=False),
        out_type=jax.ShapeDtypeStruct((B, P), jnp.float32),
        scratch_types=[
            pltpu.VMEM((NCH, WROWS), jnp.int32),
            pltpu.VMEM((2, CHUNK * S, C), jnp.float32),
            pltpu.VMEM((2, B * WROWS, C), jnp.float32),
            pltpu.VMEM((PW,), jnp.float32),
            pltpu.VMEM((B, PW), jnp.float32),
            pltpu.SemaphoreType.DMA,
            pltpu.SemaphoreType.DMA,
        ],
    )
    def run(gidx_h, w_h, bias_h, y_h, out_h,
            idx_v, w_buf, rows_buf, bias_v, out_v, sem_a, sem_b):
        cid = lax.axis_index("c")
        sid = lax.axis_index("s")
        wid = sid * NC + cid

        pltpu.sync_copy(gidx_h.at[pl.ds(wid * NCH, NCH)], idx_v)
        pltpu.sync_copy(bias_h.at[pl.ds(wid * PW, PW)], bias_v)

        sems = (sem_a, sem_b)

        def issue(ch, slot):
            for p in range(CHUNK):
                pltpu.async_copy(
                    w_h.at[(wid * NCH + ch) * CHUNK + p],
                    w_buf.at[slot].at[pl.ds(p * S, S)],
                    sems[slot])
            for b in range(B):
                pltpu.async_copy(
                    y_h.at[b].at[idx_v.at[ch]],
                    rows_buf.at[slot].at[pl.ds(b * WROWS, WROWS)],
                    sems[slot])

        def wait_chunk(ch, slot):
            for p in range(CHUNK):
                pltpu.make_async_copy(
                    w_h.at[(wid * NCH + ch) * CHUNK + p],
                    w_buf.at[slot].at[pl.ds(p * S, S)],
                    sems[slot]).wait()
            for b in range(B):
                pltpu.make_async_copy(
                    y_h.at[b].at[idx_v.at[ch]],
                    rows_buf.at[slot].at[pl.ds(b * WROWS, WROWS)],
                    sems[slot]).wait()

        iota16 = lax.iota(jnp.int32, _LANES)
        zeros = jnp.zeros((_LANES,), jnp.float32)

        issue(0, 0)

        def pair_body(t, packs):
            lbase = (t & 1) * (2 * CHUNK)

            for half in range(2):
                ch = 2 * t + half
                slot = half

                @pl.when(ch + 1 < NCH)
                def _():
                    issue(ch + 1, 1 - slot)

                wait_chunk(ch, slot)

                def patch_loop(p, packs, slot=slot, half=half):
                    def ss_loop(ss, accs):
                        row = p * S + ss

                        def cc_loop(cc, accs):
                            off = pl.ds(cc * _LANES, _LANES)
                            w = w_buf[slot, row, off]
                            return tuple(
                                accs[b] + w * rows_buf[slot, b * WROWS + row, off]
                                for b in range(B)
                            )

                        return lax.fori_loop(0, CVECS, cc_loop, accs,
                                             unroll=4)

                    accs = lax.fori_loop(0, S, ss_loop, (zeros,) * B)
                    lane = lbase + half * CHUNK + p
                    return tuple(
                        jnp.where(iota16 == lane,
                                  lax.broadcast(jnp.sum(accs[b]), (_LANES,)),
                                  packs[b])
                        for b in range(B)
                    )

                packs = lax.fori_loop(0, CHUNK, patch_loop, packs)

            flush = (t & 1) == 1
            quad = t >> 1

            @pl.when(flush)
            def _():
                bseg = bias_v[pl.ds(quad * _LANES, _LANES)]
                for b in range(B):
                    o = packs[b] + bseg
                    out_v[b, pl.ds(quad * _LANES, _LANES)] = (
                        jnp.where(o >= 0.0, o, 0.1 * o))

            return tuple(jnp.where(flush, zeros, pk) for pk in packs)

        lax.fori_loop(0, NCH // 2, pair_body, (zeros,) * B)

        for b in range(B):
            pltpu.sync_copy(out_v.at[b], out_h.at[b].at[pl.ds(wid * PW, PW)])

    return run


def kernel(y, patches, W_MULT, W_BIAS):
    B, L, C = y.shape
    P, S, _ = patches.shape

    info = plsc.get_sparse_core_info()
    NC, NS = info.num_cores, info.num_subcores

    w2 = W_MULT.reshape(P, S, C)
    gidx = patches[:, :, 0].astype(jnp.int32).reshape(P // 4, 4 * S)
    bias = W_BIAS.reshape(P)

    run = _build_sc_kernel(B, L, C, P, S, NC, NS)
    return run(gidx, w2, bias, y)
